# baseline (speedup 1.0000x reference)
import jax
import jax.numpy as jnp
from jax import lax
from jax.experimental import pallas as pl
from jax.experimental.pallas import tpu as pltpu

_MIB = 1 << 20
_DN = (((1,), (1,)), ((), ()))


def _make_body(tm, tn, in_size, n_chunks):
    crows = tn // n_chunks

    def body(x_ref, w_ref, b_ref, o_ref, wb_ref, stage_ref, sem_ref):
        j = pl.program_id(0)
        i = pl.program_id(1)

        def start(c):
            pltpu.make_async_copy(
                w_ref.at[pl.ds(j * tn + c * crows, crows), :],
                stage_ref.at[c % 2],
                sem_ref.at[c % 2],
            ).start()

        def wait(c):
            pltpu.make_async_copy(
                w_ref.at[pl.ds(0, crows), :],
                stage_ref.at[c % 2],
                sem_ref.at[c % 2],
            ).wait()

        @pl.when(i == 0)
        def _():
            xb = x_ref[...].astype(jnp.bfloat16)
            start(0)
            for c in range(n_chunks):
                if c + 1 < n_chunks:
                    start(c + 1)
                wait(c)
                wc = stage_ref[c % 2].astype(jnp.bfloat16)
                wb_ref[:, pl.ds(c * crows, crows)] = wc.T
                acc = lax.dot_general(
                    xb, wc, _DN, preferred_element_type=jnp.float32)
                o_ref[:, pl.ds(c * crows, crows)] = (
                    acc + b_ref[:, pl.ds(c * crows, crows)])

        @pl.when(i > 0)
        def _():
            acc = lax.dot_general(
                x_ref[...].astype(jnp.bfloat16), wb_ref[...],
                (((1,), (0,)), ((), ())),
                preferred_element_type=jnp.float32)
            o_ref[...] = acc + b_ref[...]

    return body


def kernel(x, weight, bias):
    B, in_size = x.shape
    out_size = weight.shape[0]
    b2 = bias.reshape(1, out_size)

    tn = out_size // 2
    tm = min(256, B)
    n_chunks = 16
    crows = tn // n_chunks
    grid = (2, pl.cdiv(B, tm))

    working = (
        tn * in_size * 2
        + 2 * crows * in_size * 4
        + 2 * tm * in_size * 4
        + 2 * tm * tn * 4
        + out_size * 4
    )
    return pl.pallas_call(
        _make_body(tm, tn, in_size, n_chunks),
        out_shape=jax.ShapeDtypeStruct((B, out_size), jnp.float32),
        grid_spec=pltpu.PrefetchScalarGridSpec(
            num_scalar_prefetch=0,
            grid=grid,
            in_specs=[
                pl.BlockSpec((tm, in_size), lambda j, i: (i, 0)),
                pl.BlockSpec(memory_space=pl.ANY),
                pl.BlockSpec((1, tn), lambda j, i: (0, j),
                             pipeline_mode=pl.Buffered(1)),
            ],
            out_specs=pl.BlockSpec((tm, tn), lambda j, i: (i, j)),
            scratch_shapes=[
                pltpu.VMEM((in_size, tn), jnp.bfloat16),
                pltpu.VMEM((2, crows, in_size), jnp.float32),
                pltpu.SemaphoreType.DMA((2,)),
            ],
        ),
        compiler_params=pltpu.CompilerParams(
            dimension_semantics=("parallel", "arbitrary"),
            vmem_limit_bytes=int(min(working + 8 * _MIB, 62 * _MIB)),
        ),
        cost_estimate=pl.CostEstimate(
            flops=2 * B * in_size * out_size,
            transcendentals=0,
            bytes_accessed=4 * (2 * B * in_size + out_size * in_size
                                + B * out_size + out_size),
        ),
    )(x, weight, b2)


# speedup vs baseline: 1.1092x; 1.1092x over previous
import jax
import jax.numpy as jnp
from jax import lax
from jax.experimental import pallas as pl
from jax.experimental.pallas import tpu as pltpu

_MIB = 1 << 20
_DN = (((1,), (1,)), ((), ()))


def _make_body(tm, tn, in_size, n_chunks):
    crows = tn // n_chunks

    def body(x_ref, w_ref, b_ref, o_ref, wb_ref, stage_ref, sem_ref):
        j = pl.program_id(0)
        i = pl.program_id(1)

        def start(c):
            pltpu.make_async_copy(
                w_ref.at[pl.ds(j * tn + c * crows, crows), :],
                stage_ref.at[c % 2],
                sem_ref.at[c % 2],
            ).start()

        def wait(c):
            pltpu.make_async_copy(
                w_ref.at[pl.ds(0, crows), :],
                stage_ref.at[c % 2],
                sem_ref.at[c % 2],
            ).wait()

        @pl.when(i == 0)
        def _():
            xb = x_ref[...].astype(jnp.bfloat16)
            start(0)
            for c in range(n_chunks):
                if c + 1 < n_chunks:
                    start(c + 1)
                wait(c)
                wc = stage_ref[c % 2].astype(jnp.bfloat16)
                wb_ref[:, pl.ds(c * crows, crows)] = wc.T
                acc = lax.dot_general(
                    xb, wc, _DN, preferred_element_type=jnp.float32)
                o_ref[:, pl.ds(c * crows, crows)] = (
                    acc + b_ref[:, pl.ds(c * crows, crows)])

        @pl.when(i > 0)
        def _():
            acc = lax.dot_general(
                x_ref[...].astype(jnp.bfloat16), wb_ref[...],
                (((1,), (0,)), ((), ())),
                preferred_element_type=jnp.float32)
            o_ref[...] = acc + b_ref[...]

    return body


def kernel(x, weight, bias):
    B, in_size = x.shape
    out_size = weight.shape[0]
    b2 = bias.reshape(1, out_size)

    tn = out_size // 2
    tm = min(256, B)
    n_chunks = 4
    crows = tn // n_chunks
    grid = (2, pl.cdiv(B, tm))

    working = (
        tn * in_size * 2
        + 2 * crows * in_size * 4
        + 2 * tm * in_size * 4
        + 2 * tm * tn * 4
        + out_size * 4
    )
    return pl.pallas_call(
        _make_body(tm, tn, in_size, n_chunks),
        out_shape=jax.ShapeDtypeStruct((B, out_size), jnp.float32),
        grid_spec=pltpu.PrefetchScalarGridSpec(
            num_scalar_prefetch=0,
            grid=grid,
            in_specs=[
                pl.BlockSpec((tm, in_size), lambda j, i: (i, 0)),
                pl.BlockSpec(memory_space=pl.ANY),
                pl.BlockSpec((1, tn), lambda j, i: (0, j),
                             pipeline_mode=pl.Buffered(1)),
            ],
            out_specs=pl.BlockSpec((tm, tn), lambda j, i: (i, j)),
            scratch_shapes=[
                pltpu.VMEM((in_size, tn), jnp.bfloat16),
                pltpu.VMEM((2, crows, in_size), jnp.float32),
                pltpu.SemaphoreType.DMA((2,)),
            ],
        ),
        compiler_params=pltpu.CompilerParams(
            dimension_semantics=("parallel", "arbitrary"),
            vmem_limit_bytes=int(min(working + 8 * _MIB, 62 * _MIB)),
        ),
        cost_estimate=pl.CostEstimate(
            flops=2 * B * in_size * out_size,
            transcendentals=0,
            bytes_accessed=4 * (2 * B * in_size + out_size * in_size
                                + B * out_size + out_size),
        ),
    )(x, weight, b2)
